# baseline (speedup 1.0000x reference)
import functools

import jax
import jax.numpy as jnp
from jax import lax
from jax.experimental import pallas as pl
from jax.experimental.pallas import tpu as pltpu
from jax.experimental.pallas import tpu_sc as plsc

_P0 = 49152
_P1 = 12288
_DEG = 8
_B = 4
_LANES = 16


def _make_sc_lap(P, D):
    ncores, nsub = 2, 16
    nw = ncores * nsub
    nodes_w = P // nw
    e_w = nodes_w * _DEG
    g_w = e_w // 128
    ec = 32768 // D
    cn = ec // _DEG
    nch = nodes_w // cn
    gpc = ec // 128
    nv = D // _LANES

    npr = 128 // D if D < 128 else 1
    ovr = cn // npr
    mesh = plsc.VectorSubcoreMesh(core_axis_name="c", subcore_axis_name="s",
                                  num_cores=ncores, num_subcores=nsub)
    out_type = jax.ShapeDtypeStruct((P * D // 128, 128), jnp.float32)
    scratch = [
        pltpu.VMEM((g_w, 128), jnp.int32),
        pltpu.VMEM((e_w + 8,), jnp.float32),
        pltpu.VMEM((2, ec, D), jnp.float32),
        pltpu.VMEM((ovr, 128), jnp.float32),
        pltpu.SemaphoreType.DMA,
        pltpu.SemaphoreType.DMA,
    ]

    def body(table, src, w, out, idx_v, w_v, rows_v, out_v, sem0, sem1):
        wid = lax.axis_index("s") * ncores + lax.axis_index("c")
        nbase = wid * nodes_w
        pltpu.sync_copy(src.at[pl.ds(wid * g_w, g_w)], idx_v)
        pltpu.sync_copy(w.at[pl.ds(wid * e_w, e_w)], w_v.at[pl.ds(0, e_w)])
        sems = [sem0, sem1]

        def issue(c, slot):
            descs = []
            for g in range(gpc):
                descs.append(pltpu.async_copy(
                    table.at[idx_v.at[c * gpc + g]],
                    rows_v.at[slot, pl.ds(g * 128, 128)],
                    sems[slot]))
            return descs

        pending = {0: issue(0, 0)}
        for c in range(nch):
            slot = c % 2
            if c + 1 < nch:
                pending[c + 1] = issue(c + 1, (c + 1) % 2)
            for dsc in pending.pop(c):
                dsc.wait()

            def row_body(m, carry, slot=slot, c=c):
                for par in range(npr):
                    n = m * npr + par
                    e0 = n * _DEG
                    w16 = w_v[pl.ds(c * ec + e0, _LANES)]
                    accs = [jnp.zeros((_LANES,), jnp.float32)
                            for _ in range(nv)]
                    for d in range(_DEG):
                        wv = w16[d]
                        for j in range(nv):
                            row = rows_v[slot, e0 + d,
                                         pl.ds(j * _LANES, _LANES)]
                            accs[j] = accs[j] + row * wv
                    for j in range(nv):
                        out_v[m, pl.ds(par * D + j * _LANES, _LANES)] = accs[j]
                return carry

            lax.fori_loop(0, ovr, row_body, 0)
            pltpu.sync_copy(
                out_v, out.at[pl.ds((nbase + c * cn) // npr, ovr)])

    return pl.kernel(body, out_type=out_type, mesh=mesh,
                     scratch_types=scratch,
                     compiler_params=pltpu.CompilerParams(
                         use_tc_tiling_on_sc=False))


def _ln_relu_lanes(y, s, g, be):
    m = jnp.dot(y, s, preferred_element_type=jnp.float32)
    q = jnp.dot(y * y, s, preferred_element_type=jnp.float32)
    v = q - m * m
    return jnp.maximum((y - m) * lax.rsqrt(v + 1e-5) * g + be, 0.0)


def _combine0(x0, x1, x2, k0, k1, k2, s, b, g, be):
    tp = 512

    def body(x0r, x1r, x2r, k0r, k1r, k2r, sr, br, gr, ber, outr):
        y = (jnp.dot(x0r[...], k0r[...], preferred_element_type=jnp.float32)
             + jnp.dot(x1r[...], k1r[...], preferred_element_type=jnp.float32)
             + jnp.dot(x2r[...], k2r[...], preferred_element_type=jnp.float32)
             + br[...])
        y = _ln_relu_lanes(y, sr[...], gr[...], ber[...])
        ys = y.reshape(tp // 2, 2, 2, 128)
        outr[...] = (ys[:, 0, 0] + ys[:, 0, 1]
                     + ys[:, 1, 0] + ys[:, 1, 1]) * 0.25

    grid = _P0 // 2 // tp
    return pl.pallas_call(
        body,
        grid=(grid,),
        in_specs=[
            pl.BlockSpec((tp, 128), lambda i: (i, 0)),
            pl.BlockSpec((tp, 128), lambda i: (i, 0)),
            pl.BlockSpec((tp, 128), lambda i: (i, 0)),
            pl.BlockSpec((128, 256), lambda i: (0, 0)),
            pl.BlockSpec((128, 256), lambda i: (0, 0)),
            pl.BlockSpec((128, 256), lambda i: (0, 0)),
            pl.BlockSpec((256, 256), lambda i: (0, 0)),
            pl.BlockSpec((1, 256), lambda i: (0, 0)),
            pl.BlockSpec((1, 256), lambda i: (0, 0)),
            pl.BlockSpec((1, 256), lambda i: (0, 0)),
        ],
        out_specs=pl.BlockSpec((tp // 2, 128), lambda i: (i, 0)),
        out_shape=jax.ShapeDtypeStruct((_P1, 128), jnp.float32),
    )(x0, x1, x2, k0, k1, k2, s, b, g, be)


def _combine1_head(x0, x1, x2, k0, k1, k2, s, b, g, be, wh, bh, wo, bo):
    tp = 1024
    grid = _P1 // tp

    def body(x0r, x1r, x2r, k0r, k1r, k2r, sr, br, gr, ber,
             whr, bhr, wor, bor, outr, zsum):
        y = (jnp.dot(x0r[...], k0r[...], preferred_element_type=jnp.float32)
             + jnp.dot(x1r[...], k1r[...], preferred_element_type=jnp.float32)
             + jnp.dot(x2r[...], k2r[...], preferred_element_type=jnp.float32)
             + br[...])
        y = _ln_relu_lanes(y, sr[...], gr[...], ber[...])
        part = jnp.sum(y, axis=0, keepdims=True)

        @pl.when(pl.program_id(0) == 0)
        def _():
            zsum[...] = jnp.zeros_like(zsum)

        zsum[...] += part

        @pl.when(pl.program_id(0) == grid - 1)
        def _():
            z = zsum[...] * (1.0 / _P1)
            h = jnp.maximum(
                jnp.dot(z, whr[...], preferred_element_type=jnp.float32)
                + bhr[...], 0.0)
            outr[...] = (jnp.dot(h, wor[...],
                                 preferred_element_type=jnp.float32)
                         + bor[...])

    return pl.pallas_call(
        body,
        grid=(grid,),
        in_specs=[
            pl.BlockSpec((tp, 128), lambda i: (i, 0)),
            pl.BlockSpec((tp, 128), lambda i: (i, 0)),
            pl.BlockSpec((tp, 128), lambda i: (i, 0)),
            pl.BlockSpec((128, 128), lambda i: (0, 0)),
            pl.BlockSpec((128, 128), lambda i: (0, 0)),
            pl.BlockSpec((128, 128), lambda i: (0, 0)),
            pl.BlockSpec((128, 128), lambda i: (0, 0)),
            pl.BlockSpec((1, 128), lambda i: (0, 0)),
            pl.BlockSpec((1, 128), lambda i: (0, 0)),
            pl.BlockSpec((1, 128), lambda i: (0, 0)),
            pl.BlockSpec((128, _B * 256), lambda i: (0, 0)),
            pl.BlockSpec((1, _B * 256), lambda i: (0, 0)),
            pl.BlockSpec((_B * 256, _B * 128), lambda i: (0, 0)),
            pl.BlockSpec((1, _B * 128), lambda i: (0, 0)),
        ],
        out_specs=pl.BlockSpec((1, _B * 128), lambda i: (0, 0)),
        out_shape=jax.ShapeDtypeStruct((1, _B * 128), jnp.float32),
        scratch_shapes=[pltpu.VMEM((1, 128), jnp.float32)],
    )(x0, x1, x2, k0, k1, k2, s, b, g, be, wh, bh, wo, bo)


def kernel(x, ei0, w0_lap, ei1, w1_lap, cW0, cb0, g0, be0,
           cW1, cb1, g1, be1, Wh, bh, Wo, bo):
    lap0 = _make_sc_lap(_P0, _B * 16)
    lap1 = _make_sc_lap(_P1, _B * 32)

    xt2 = jnp.transpose(x, (1, 0, 2)).reshape(_P0 // 2, 128)
    xt = xt2.reshape(_P0, _B * 16)
    src0 = ei0[0].reshape(-1, 128)
    src1 = ei1[0].reshape(-1, 128)

    x1t = lap0(xt, src0, w0_lap)
    y2t = lap0(x1t.reshape(_P0, _B * 16), src0, w0_lap)

    eye = jnp.eye(_B, dtype=jnp.float32)
    eye8 = jnp.eye(2 * _B, dtype=jnp.float32)
    seg32 = jnp.kron(eye, jnp.full((32, 32), 1.0 / 32, jnp.float32))
    seg32x8 = jnp.kron(eye8, jnp.full((32, 32), 1.0 / 32, jnp.float32))

    h0t = _combine0(
        xt2, x1t, y2t,
        jnp.kron(eye8, cW0[0] - cW0[2]), jnp.kron(eye8, cW0[1]),
        jnp.kron(eye8, 2.0 * cW0[2]), seg32x8,
        jnp.tile(cb0, 2 * _B).reshape(1, -1),
        jnp.tile(g0, 2 * _B).reshape(1, -1),
        jnp.tile(be0, 2 * _B).reshape(1, -1))

    h1t = lap1(h0t, src1, w1_lap)
    y2t1 = lap1(h1t, src1, w1_lap)

    out = _combine1_head(
        h0t, h1t, y2t1,
        jnp.kron(eye, cW1[0] - cW1[2]), jnp.kron(eye, cW1[1]),
        jnp.kron(eye, 2.0 * cW1[2]), seg32,
        jnp.tile(cb1, _B).reshape(1, -1), jnp.tile(g1, _B).reshape(1, -1),
        jnp.tile(be1, _B).reshape(1, -1),
        jnp.kron(eye, Wh), jnp.tile(bh, _B).reshape(1, -1),
        jnp.kron(eye, Wo), jnp.tile(bo, _B).reshape(1, -1))
    return out.reshape(_B, 128)


# speedup vs baseline: 1.1230x; 1.1230x over previous
import functools

import jax
import jax.numpy as jnp
from jax import lax
from jax.experimental import pallas as pl
from jax.experimental.pallas import tpu as pltpu
from jax.experimental.pallas import tpu_sc as plsc

_P0 = 49152
_P1 = 12288
_DEG = 8
_B = 4
_LANES = 16


def _make_sc_lap(P, D):
    ncores, nsub = 2, 16
    nw = ncores * nsub
    nodes_w = P // nw
    e_w = nodes_w * _DEG
    g_w = e_w // 128
    ec = 32768 // D
    cn = ec // _DEG
    nch = nodes_w // cn
    gpc = ec // 128
    nv = D // _LANES

    npr = 128 // D if D < 128 else 1
    ovr = cn // npr
    mesh = plsc.VectorSubcoreMesh(core_axis_name="c", subcore_axis_name="s",
                                  num_cores=ncores, num_subcores=nsub)
    out_type = jax.ShapeDtypeStruct((P * D // 128, 128), jnp.float32)
    scratch = [
        pltpu.VMEM((g_w, 128), jnp.int32),
        pltpu.VMEM((e_w + 8,), jnp.float32),
        pltpu.VMEM((2, ec, D), jnp.float32),
        pltpu.VMEM((ovr, 128), jnp.float32),
        pltpu.SemaphoreType.DMA,
        pltpu.SemaphoreType.DMA,
    ]

    def body(table, src, w, out, idx_v, w_v, rows_v, out_v, sem0, sem1):
        wid = lax.axis_index("s") * ncores + lax.axis_index("c")
        nbase = wid * nodes_w
        pltpu.sync_copy(src.at[pl.ds(wid * g_w, g_w)], idx_v)
        pltpu.sync_copy(w.at[pl.ds(wid * e_w, e_w)], w_v.at[pl.ds(0, e_w)])
        sems = [sem0, sem1]

        def issue(c, slot):
            descs = []
            for g in range(gpc):
                descs.append(pltpu.async_copy(
                    table.at[idx_v.at[c * gpc + g]],
                    rows_v.at[slot, pl.ds(g * 128, 128)],
                    sems[slot]))
            return descs

        pending = {0: issue(0, 0)}
        for c in range(nch):
            slot = c % 2
            if c + 1 < nch:
                pending[c + 1] = issue(c + 1, (c + 1) % 2)
            for dsc in pending.pop(c):
                dsc.wait()

            def row_body(m, carry, slot=slot, c=c):
                for par in range(npr):
                    n = m * npr + par
                    e0 = n * _DEG
                    w16 = w_v[pl.ds(c * ec + e0, _LANES)]
                    accs = [jnp.zeros((_LANES,), jnp.float32)
                            for _ in range(nv)]
                    for d in range(_DEG):
                        wv = w16[d]
                        for j in range(nv):
                            row = rows_v[slot, e0 + d,
                                         pl.ds(j * _LANES, _LANES)]
                            accs[j] = accs[j] + row * wv
                    for j in range(nv):
                        out_v[m, pl.ds(par * D + j * _LANES, _LANES)] = accs[j]
                return carry

            lax.fori_loop(0, ovr, row_body, 0)
            pltpu.sync_copy(
                out_v, out.at[pl.ds((nbase + c * cn) // npr, ovr)])

    return pl.kernel(body, out_type=out_type, mesh=mesh,
                     scratch_types=scratch,
                     compiler_params=pltpu.CompilerParams(
                         use_tc_tiling_on_sc=False))


def _ln_relu_lanes(y, s, g, be):
    m = jnp.dot(y, s, preferred_element_type=jnp.float32)
    q = jnp.dot(y * y, s, preferred_element_type=jnp.float32)
    v = q - m * m
    return jnp.maximum((y - m) * lax.rsqrt(v + 1e-5) * g + be, 0.0)


def _combine0(x0, x1, x2, k0, k1, k2, s, b, g, be):
    tp = 1024

    def body(x0r, x1r, x2r, k0r, k1r, k2r, sr, br, gr, ber, outr):
        y0 = jnp.dot(x0r[...], k0r[...], preferred_element_type=jnp.float32)
        y = (y0.reshape(tp // 2, 256)
             + jnp.dot(x1r[...], k1r[...], preferred_element_type=jnp.float32)
             + jnp.dot(x2r[...], k2r[...], preferred_element_type=jnp.float32)
             + br[...])
        y = _ln_relu_lanes(y, sr[...], gr[...], ber[...])
        ys = y.reshape(tp // 4, 2, 2, 128)
        outr[...] = (ys[:, 0, 0] + ys[:, 0, 1]
                     + ys[:, 1, 0] + ys[:, 1, 1]) * 0.25

    grid = _P0 // tp
    return pl.pallas_call(
        body,
        grid=(grid,),
        in_specs=[
            pl.BlockSpec((tp, 64), lambda i: (i, 0)),
            pl.BlockSpec((tp // 2, 128), lambda i: (i, 0)),
            pl.BlockSpec((tp // 2, 128), lambda i: (i, 0)),
            pl.BlockSpec((64, 128), lambda i: (0, 0)),
            pl.BlockSpec((128, 256), lambda i: (0, 0)),
            pl.BlockSpec((128, 256), lambda i: (0, 0)),
            pl.BlockSpec((256, 256), lambda i: (0, 0)),
            pl.BlockSpec((1, 256), lambda i: (0, 0)),
            pl.BlockSpec((1, 256), lambda i: (0, 0)),
            pl.BlockSpec((1, 256), lambda i: (0, 0)),
        ],
        out_specs=pl.BlockSpec((tp // 4, 128), lambda i: (i, 0)),
        out_shape=jax.ShapeDtypeStruct((_P1, 128), jnp.float32),
    )(x0, x1, x2, k0, k1, k2, s, b, g, be)


def _combine1_head(x0, x1, x2, k0, k1, k2, s, b, g, be, wh, bh, wo, bo):
    tp = 1024
    grid = _P1 // tp

    def body(x0r, x1r, x2r, k0r, k1r, k2r, sr, br, gr, ber,
             whr, bhr, wor, bor, outr, zsum):
        y = (jnp.dot(x0r[...], k0r[...], preferred_element_type=jnp.float32)
             + jnp.dot(x1r[...], k1r[...], preferred_element_type=jnp.float32)
             + jnp.dot(x2r[...], k2r[...], preferred_element_type=jnp.float32)
             + br[...])
        y = _ln_relu_lanes(y, sr[...], gr[...], ber[...])
        part = jnp.sum(y, axis=0, keepdims=True)

        @pl.when(pl.program_id(0) == 0)
        def _():
            zsum[...] = jnp.zeros_like(zsum)

        zsum[...] += part

        @pl.when(pl.program_id(0) == grid - 1)
        def _():
            z = zsum[...] * (1.0 / _P1)
            h = jnp.maximum(
                jnp.dot(z, whr[...], preferred_element_type=jnp.float32)
                + bhr[...], 0.0)
            outr[...] = (jnp.dot(h, wor[...],
                                 preferred_element_type=jnp.float32)
                         + bor[...])

    return pl.pallas_call(
        body,
        grid=(grid,),
        in_specs=[
            pl.BlockSpec((tp, 128), lambda i: (i, 0)),
            pl.BlockSpec((tp, 128), lambda i: (i, 0)),
            pl.BlockSpec((tp, 128), lambda i: (i, 0)),
            pl.BlockSpec((128, 128), lambda i: (0, 0)),
            pl.BlockSpec((128, 128), lambda i: (0, 0)),
            pl.BlockSpec((128, 128), lambda i: (0, 0)),
            pl.BlockSpec((128, 128), lambda i: (0, 0)),
            pl.BlockSpec((1, 128), lambda i: (0, 0)),
            pl.BlockSpec((1, 128), lambda i: (0, 0)),
            pl.BlockSpec((1, 128), lambda i: (0, 0)),
            pl.BlockSpec((128, _B * 256), lambda i: (0, 0)),
            pl.BlockSpec((1, _B * 256), lambda i: (0, 0)),
            pl.BlockSpec((_B * 256, _B * 128), lambda i: (0, 0)),
            pl.BlockSpec((1, _B * 128), lambda i: (0, 0)),
        ],
        out_specs=pl.BlockSpec((1, _B * 128), lambda i: (0, 0)),
        out_shape=jax.ShapeDtypeStruct((1, _B * 128), jnp.float32),
        scratch_shapes=[pltpu.VMEM((1, 128), jnp.float32)],
    )(x0, x1, x2, k0, k1, k2, s, b, g, be, wh, bh, wo, bo)


def kernel(x, ei0, w0_lap, ei1, w1_lap, cW0, cb0, g0, be0,
           cW1, cb1, g1, be1, Wh, bh, Wo, bo):
    lap0 = _make_sc_lap(_P0, _B * 16)
    lap1 = _make_sc_lap(_P1, _B * 32)

    xt = jnp.transpose(x, (1, 0, 2)).reshape(_P0, _B * 16)
    src0 = ei0[0].reshape(-1, 128)
    src1 = ei1[0].reshape(-1, 128)

    x1t = lap0(xt, src0, w0_lap)
    y2t = lap0(x1t.reshape(_P0, _B * 16), src0, w0_lap)

    eye = jnp.eye(_B, dtype=jnp.float32)
    eye8 = jnp.eye(2 * _B, dtype=jnp.float32)
    seg32 = jnp.kron(eye, jnp.full((32, 32), 1.0 / 32, jnp.float32))
    seg32x8 = jnp.kron(eye8, jnp.full((32, 32), 1.0 / 32, jnp.float32))

    h0t = _combine0(
        xt, x1t, y2t,
        jnp.kron(eye, cW0[0] - cW0[2]), jnp.kron(eye8, cW0[1]),
        jnp.kron(eye8, 2.0 * cW0[2]), seg32x8,
        jnp.tile(cb0, 2 * _B).reshape(1, -1),
        jnp.tile(g0, 2 * _B).reshape(1, -1),
        jnp.tile(be0, 2 * _B).reshape(1, -1))

    h1t = lap1(h0t, src1, w1_lap)
    y2t1 = lap1(h1t, src1, w1_lap)

    out = _combine1_head(
        h0t, h1t, y2t1,
        jnp.kron(eye, cW1[0] - cW1[2]), jnp.kron(eye, cW1[1]),
        jnp.kron(eye, 2.0 * cW1[2]), seg32,
        jnp.tile(cb1, _B).reshape(1, -1), jnp.tile(g1, _B).reshape(1, -1),
        jnp.tile(be1, _B).reshape(1, -1),
        jnp.kron(eye, Wh), jnp.tile(bh, _B).reshape(1, -1),
        jnp.kron(eye, Wo), jnp.tile(bo, _B).reshape(1, -1))
    return out.reshape(_B, 128)


# speedup vs baseline: 1.1451x; 1.0197x over previous
import functools

import jax
import jax.numpy as jnp
from jax import lax
from jax.experimental import pallas as pl
from jax.experimental.pallas import tpu as pltpu
from jax.experimental.pallas import tpu_sc as plsc

_P0 = 49152
_P1 = 12288
_DEG = 8
_B = 4
_LANES = 16


def _make_sc_lap(P, D, table_packed=False, emit_packed=False):
    ncores, nsub = 2, 16
    nw = ncores * nsub
    nodes_w = P // nw
    e_w = nodes_w * _DEG
    g_w = e_w // 128
    dw = D // 2 if table_packed else D
    ec = 32768 // dw
    cn = ec // _DEG
    nch = nodes_w // cn
    gpc = ec // 128
    nv = D // _LANES

    npr = 128 // D if D < 128 else 1
    ovr = cn // npr
    nprb = 256 // D
    ovrb = cn // nprb
    mesh = plsc.VectorSubcoreMesh(core_axis_name="c", subcore_axis_name="s",
                                  num_cores=ncores, num_subcores=nsub)
    out_type = [jax.ShapeDtypeStruct((P * D // 128, 128), jnp.float32)]
    if emit_packed:
        out_type.append(jax.ShapeDtypeStruct((P * D // 256, 128),
                                             jnp.float32))
    scratch = [
        pltpu.VMEM((g_w, 128), jnp.int32),
        pltpu.VMEM((e_w + 8,), jnp.float32),
        pltpu.VMEM((2, ec, dw), jnp.float32),
        pltpu.VMEM((ovr, 128), jnp.float32),
        pltpu.VMEM((max(ovrb, 1), 128), jnp.float32),
        pltpu.SemaphoreType.DMA,
        pltpu.SemaphoreType.DMA,
    ]

    def body(*refs):
        if emit_packed:
            (table, src, w, out, outb, idx_v, w_v, rows_v, out_v, out_vb,
             sem0, sem1) = refs
        else:
            (table, src, w, out, idx_v, w_v, rows_v, out_v, out_vb,
             sem0, sem1) = refs
            outb = None
        wid = lax.axis_index("s") * ncores + lax.axis_index("c")
        nbase = wid * nodes_w
        pltpu.sync_copy(src.at[pl.ds(wid * g_w, g_w)], idx_v)
        pltpu.sync_copy(w.at[pl.ds(wid * e_w, e_w)], w_v.at[pl.ds(0, e_w)])
        sems = [sem0, sem1]

        def issue(c, slot):
            descs = []
            for g in range(gpc):
                descs.append(pltpu.async_copy(
                    table.at[idx_v.at[c * gpc + g]],
                    rows_v.at[slot, pl.ds(g * 128, 128)],
                    sems[slot]))
            return descs

        def accumulate(slot, c, n):
            e0 = n * _DEG
            w16 = w_v[pl.ds(c * ec + e0, _LANES)]
            accs = [jnp.zeros((_LANES,), jnp.float32) for _ in range(nv)]
            for d in range(_DEG):
                wv = w16[d]
                if table_packed:
                    for j in range(nv // 2):
                        pr = rows_v[slot, e0 + d, pl.ds(j * _LANES, _LANES)]
                        a, b = plsc.unpack(
                            plsc.bitcast(pr, jnp.bfloat16),
                            format=plsc.PackFormat.INTERLEAVED)
                        accs[2 * j] = accs[2 * j] + a * wv
                        accs[2 * j + 1] = accs[2 * j + 1] + b * wv
                else:
                    for j in range(nv):
                        row = rows_v[slot, e0 + d, pl.ds(j * _LANES, _LANES)]
                        accs[j] = accs[j] + row * wv
            return accs

        pending = {0: issue(0, 0)}
        for c in range(nch):
            slot = c % 2
            if c + 1 < nch:
                pending[c + 1] = issue(c + 1, (c + 1) % 2)
            for dsc in pending.pop(c):
                dsc.wait()

            def row_body(m, carry, slot=slot, c=c):
                for kk in range(nprb):
                    n = m * nprb + kk
                    accs = accumulate(slot, c, n)
                    mr = m * (nprb // npr) + kk // npr
                    for j in range(nv):
                        out_v[mr, pl.ds((n % npr) * D + j * _LANES,
                                        _LANES)] = accs[j]
                    if emit_packed:
                        for j in range(nv // 2):
                            pk = plsc.bitcast(
                                plsc.pack(accs[2 * j], accs[2 * j + 1],
                                          format=plsc.PackFormat.INTERLEAVED),
                                jnp.float32)
                            out_vb[m, pl.ds(kk * (D // 2) + j * _LANES,
                                            _LANES)] = pk
                return carry

            lax.fori_loop(0, ovrb, row_body, 0)
            pltpu.sync_copy(
                out_v, out.at[pl.ds((nbase + c * cn) // npr, ovr)])
            if emit_packed:
                pltpu.sync_copy(
                    out_vb, outb.at[pl.ds((nbase + c * cn) // nprb, ovrb)])

    return pl.kernel(body, out_type=tuple(out_type) if emit_packed
                     else out_type[0],
                     mesh=mesh, scratch_types=scratch,
                     compiler_params=pltpu.CompilerParams(
                         use_tc_tiling_on_sc=False,
                         needs_layout_passes=False))


def _ln_relu_lanes(y, s, g, be):
    m = jnp.dot(y, s, preferred_element_type=jnp.float32)
    q = jnp.dot(y * y, s, preferred_element_type=jnp.float32)
    v = q - m * m
    return jnp.maximum((y - m) * lax.rsqrt(v + 1e-5) * g + be, 0.0)


def _combine0(x0, x1, x2, k0, k1, k2, s, b, g, be):
    tp = 1024

    def body(x0r, x1r, x2r, k0r, k1r, k2r, sr, br, gr, ber, outr):
        y0 = jnp.dot(x0r[...], k0r[...], preferred_element_type=jnp.float32)
        y = (y0.reshape(tp // 2, 256)
             + jnp.dot(x1r[...], k1r[...], preferred_element_type=jnp.float32)
             + jnp.dot(x2r[...], k2r[...], preferred_element_type=jnp.float32)
             + br[...])
        y = _ln_relu_lanes(y, sr[...], gr[...], ber[...])
        ys = y.reshape(tp // 4, 2, 2, 128)
        outr[...] = (ys[:, 0, 0] + ys[:, 0, 1]
                     + ys[:, 1, 0] + ys[:, 1, 1]) * 0.25

    grid = _P0 // tp
    return pl.pallas_call(
        body,
        grid=(grid,),
        in_specs=[
            pl.BlockSpec((tp, 64), lambda i: (i, 0)),
            pl.BlockSpec((tp // 2, 128), lambda i: (i, 0)),
            pl.BlockSpec((tp // 2, 128), lambda i: (i, 0)),
            pl.BlockSpec((64, 128), lambda i: (0, 0)),
            pl.BlockSpec((128, 256), lambda i: (0, 0)),
            pl.BlockSpec((128, 256), lambda i: (0, 0)),
            pl.BlockSpec((256, 256), lambda i: (0, 0)),
            pl.BlockSpec((1, 256), lambda i: (0, 0)),
            pl.BlockSpec((1, 256), lambda i: (0, 0)),
            pl.BlockSpec((1, 256), lambda i: (0, 0)),
        ],
        out_specs=pl.BlockSpec((tp // 4, 128), lambda i: (i, 0)),
        out_shape=jax.ShapeDtypeStruct((_P1, 128), jnp.float32),
    )(x0, x1, x2, k0, k1, k2, s, b, g, be)


def _combine1_head(x0, x1, x2, k0, k1, k2, s, b, g, be, wh, bh, wo, bo):
    tp = 1024
    grid = _P1 // tp

    def body(x0r, x1r, x2r, k0r, k1r, k2r, sr, br, gr, ber,
             whr, bhr, wor, bor, outr, zsum):
        y = (jnp.dot(x0r[...], k0r[...], preferred_element_type=jnp.float32)
             + jnp.dot(x1r[...], k1r[...], preferred_element_type=jnp.float32)
             + jnp.dot(x2r[...], k2r[...], preferred_element_type=jnp.float32)
             + br[...])
        y = _ln_relu_lanes(y, sr[...], gr[...], ber[...])
        part = jnp.sum(y, axis=0, keepdims=True)

        @pl.when(pl.program_id(0) == 0)
        def _():
            zsum[...] = jnp.zeros_like(zsum)

        zsum[...] += part

        @pl.when(pl.program_id(0) == grid - 1)
        def _():
            z = zsum[...] * (1.0 / _P1)
            h = jnp.maximum(
                jnp.dot(z, whr[...], preferred_element_type=jnp.float32)
                + bhr[...], 0.0)
            outr[...] = (jnp.dot(h, wor[...],
                                 preferred_element_type=jnp.float32)
                         + bor[...])

    return pl.pallas_call(
        body,
        grid=(grid,),
        in_specs=[
            pl.BlockSpec((tp, 128), lambda i: (i, 0)),
            pl.BlockSpec((tp, 128), lambda i: (i, 0)),
            pl.BlockSpec((tp, 128), lambda i: (i, 0)),
            pl.BlockSpec((128, 128), lambda i: (0, 0)),
            pl.BlockSpec((128, 128), lambda i: (0, 0)),
            pl.BlockSpec((128, 128), lambda i: (0, 0)),
            pl.BlockSpec((128, 128), lambda i: (0, 0)),
            pl.BlockSpec((1, 128), lambda i: (0, 0)),
            pl.BlockSpec((1, 128), lambda i: (0, 0)),
            pl.BlockSpec((1, 128), lambda i: (0, 0)),
            pl.BlockSpec((128, _B * 256), lambda i: (0, 0)),
            pl.BlockSpec((1, _B * 256), lambda i: (0, 0)),
            pl.BlockSpec((_B * 256, _B * 128), lambda i: (0, 0)),
            pl.BlockSpec((1, _B * 128), lambda i: (0, 0)),
        ],
        out_specs=pl.BlockSpec((1, _B * 128), lambda i: (0, 0)),
        out_shape=jax.ShapeDtypeStruct((1, _B * 128), jnp.float32),
        scratch_shapes=[pltpu.VMEM((1, 128), jnp.float32)],
    )(x0, x1, x2, k0, k1, k2, s, b, g, be, wh, bh, wo, bo)


def kernel(x, ei0, w0_lap, ei1, w1_lap, cW0, cb0, g0, be0,
           cW1, cb1, g1, be1, Wh, bh, Wo, bo):
    lap0a = _make_sc_lap(_P0, _B * 16, emit_packed=True)
    lap0b = _make_sc_lap(_P0, _B * 16, table_packed=True)
    lap1a = _make_sc_lap(_P1, _B * 32, emit_packed=True)
    lap1b = _make_sc_lap(_P1, _B * 32, table_packed=True)

    xt = jnp.transpose(x, (1, 0, 2)).reshape(_P0, _B * 16)
    src0 = ei0[0].reshape(-1, 128)
    src1 = ei1[0].reshape(-1, 128)

    x1t, x1p = lap0a(xt, src0, w0_lap)
    y2t = lap0b(x1p.reshape(_P0, _B * 8), src0, w0_lap)

    eye = jnp.eye(_B, dtype=jnp.float32)
    eye8 = jnp.eye(2 * _B, dtype=jnp.float32)
    seg32 = jnp.kron(eye, jnp.full((32, 32), 1.0 / 32, jnp.float32))
    seg32x8 = jnp.kron(eye8, jnp.full((32, 32), 1.0 / 32, jnp.float32))

    h0t = _combine0(
        xt, x1t, y2t,
        jnp.kron(eye, cW0[0] - cW0[2]), jnp.kron(eye8, cW0[1]),
        jnp.kron(eye8, 2.0 * cW0[2]), seg32x8,
        jnp.tile(cb0, 2 * _B).reshape(1, -1),
        jnp.tile(g0, 2 * _B).reshape(1, -1),
        jnp.tile(be0, 2 * _B).reshape(1, -1))

    h1t, h1p = lap1a(h0t, src1, w1_lap)
    y2t1 = lap1b(h1p.reshape(_P1, _B * 16), src1, w1_lap)

    out = _combine1_head(
        h0t, h1t, y2t1,
        jnp.kron(eye, cW1[0] - cW1[2]), jnp.kron(eye, cW1[1]),
        jnp.kron(eye, 2.0 * cW1[2]), seg32,
        jnp.tile(cb1, _B).reshape(1, -1), jnp.tile(g1, _B).reshape(1, -1),
        jnp.tile(be1, _B).reshape(1, -1),
        jnp.kron(eye, Wh), jnp.tile(bh, _B).reshape(1, -1),
        jnp.kron(eye, Wo), jnp.tile(bo, _B).reshape(1, -1))
    return out.reshape(_B, 128)


# speedup vs baseline: 1.1779x; 1.0287x over previous
import functools

import jax
import jax.numpy as jnp
from jax import lax
from jax.experimental import pallas as pl
from jax.experimental.pallas import tpu as pltpu
from jax.experimental.pallas import tpu_sc as plsc

_P0 = 49152
_P1 = 12288
_DEG = 8
_B = 4
_LANES = 16


def _make_sc_lap(P, D, table_packed=False, emit_packed=False):
    ncores, nsub = 2, 16
    nw = ncores * nsub
    nodes_w = P // nw
    e_w = nodes_w * _DEG
    g_w = e_w // 128
    dw = D // 2 if table_packed else D
    ec = 32768 // dw
    cn = ec // _DEG
    nch = nodes_w // cn
    gpc = ec // 128
    nv = D // _LANES

    npr = 128 // D if D < 128 else 1
    ovr = cn // npr
    nprb = 256 // D
    ovrb = cn // nprb
    mesh = plsc.VectorSubcoreMesh(core_axis_name="c", subcore_axis_name="s",
                                  num_cores=ncores, num_subcores=nsub)
    out_type = [jax.ShapeDtypeStruct((P * D // 128, 128), jnp.float32)]
    if emit_packed:
        out_type.append(jax.ShapeDtypeStruct((P * D // 256, 128),
                                             jnp.float32))
    scratch = [
        pltpu.VMEM((g_w, 128), jnp.int32),
        pltpu.VMEM((e_w + 8,), jnp.float32),
        pltpu.VMEM((2, ec, dw), jnp.float32),
        pltpu.VMEM((ovr, 128), jnp.float32),
        pltpu.VMEM((max(ovrb, 1), 128), jnp.float32),
        pltpu.SemaphoreType.DMA,
        pltpu.SemaphoreType.DMA,
    ]

    def body(*refs):
        if emit_packed:
            (table, src, w, out, outb, idx_v, w_v, rows_v, out_v, out_vb,
             sem0, sem1) = refs
        else:
            (table, src, w, out, idx_v, w_v, rows_v, out_v, out_vb,
             sem0, sem1) = refs
            outb = None
        wid = lax.axis_index("s") * ncores + lax.axis_index("c")
        nbase = wid * nodes_w
        pltpu.sync_copy(src.at[pl.ds(wid * g_w, g_w)], idx_v)
        pltpu.sync_copy(w.at[pl.ds(wid * e_w, e_w)], w_v.at[pl.ds(0, e_w)])
        sems = [sem0, sem1]

        def issue(c, slot):
            descs = []
            for g in range(gpc):
                descs.append(pltpu.async_copy(
                    table.at[idx_v.at[c * gpc + g]],
                    rows_v.at[slot, pl.ds(g * 128, 128)],
                    sems[slot]))
            return descs

        def accumulate(slot, c, n):
            e0 = n * _DEG
            w16 = w_v[pl.ds(c * ec + e0, _LANES)]
            accs = [jnp.zeros((_LANES,), jnp.float32) for _ in range(nv)]
            for d in range(_DEG):
                wv = w16[d]
                if table_packed:
                    for j in range(nv // 2):
                        pr = rows_v[slot, e0 + d, pl.ds(j * _LANES, _LANES)]
                        a, b = plsc.unpack(
                            plsc.bitcast(pr, jnp.bfloat16),
                            format=plsc.PackFormat.INTERLEAVED)
                        accs[2 * j] = accs[2 * j] + a * wv
                        accs[2 * j + 1] = accs[2 * j + 1] + b * wv
                else:
                    for j in range(nv):
                        row = rows_v[slot, e0 + d, pl.ds(j * _LANES, _LANES)]
                        accs[j] = accs[j] + row * wv
            return accs

        pending = {0: issue(0, 0)}
        for c in range(nch):
            slot = c % 2
            if c + 1 < nch:
                pending[c + 1] = issue(c + 1, (c + 1) % 2)
            for dsc in pending.pop(c):
                dsc.wait()

            def row_body(m, carry, slot=slot, c=c):
                for kk in range(nprb):
                    n = m * nprb + kk
                    accs = accumulate(slot, c, n)
                    mr = m * (nprb // npr) + kk // npr
                    for j in range(nv):
                        out_v[mr, pl.ds((n % npr) * D + j * _LANES,
                                        _LANES)] = accs[j]
                    if emit_packed:
                        for j in range(nv // 2):
                            pk = plsc.bitcast(
                                plsc.pack(accs[2 * j], accs[2 * j + 1],
                                          format=plsc.PackFormat.INTERLEAVED),
                                jnp.float32)
                            out_vb[m, pl.ds(kk * (D // 2) + j * _LANES,
                                            _LANES)] = pk
                return carry

            lax.fori_loop(0, ovrb, row_body, 0)
            pltpu.sync_copy(
                out_v, out.at[pl.ds((nbase + c * cn) // npr, ovr)])
            if emit_packed:
                pltpu.sync_copy(
                    out_vb, outb.at[pl.ds((nbase + c * cn) // nprb, ovrb)])

    return pl.kernel(body, out_type=tuple(out_type) if emit_packed
                     else out_type[0],
                     mesh=mesh, scratch_types=scratch,
                     compiler_params=pltpu.CompilerParams(
                         use_tc_tiling_on_sc=False,
                         needs_layout_passes=False))


def _ln_relu_lanes(y, s, g, be):
    m = jnp.dot(y, s, preferred_element_type=jnp.float32)
    q = jnp.dot(y * y, s, preferred_element_type=jnp.float32)
    v = q - m * m
    return jnp.maximum((y - m) * lax.rsqrt(v + 1e-5) * g + be, 0.0)


def _combine0(x0, x1, x2, k0, k1, k2, s, b, g, be):
    tp = 2048

    def body(x0r, x1r, x2r, k0r, k1r, k2r, sr, br, gr, ber, outr):
        y0 = jnp.dot(x0r[...], k0r[...], preferred_element_type=jnp.float32)
        y = (y0.reshape(tp // 2, 256)
             + jnp.dot(x1r[...], k1r[...], preferred_element_type=jnp.float32)
             + jnp.dot(x2r[...], k2r[...], preferred_element_type=jnp.float32)
             + br[...])
        y = _ln_relu_lanes(y, sr[...], gr[...], ber[...])
        ys = y.reshape(tp // 4, 2, 2, 128)
        outr[...] = (ys[:, 0, 0] + ys[:, 0, 1]
                     + ys[:, 1, 0] + ys[:, 1, 1]) * 0.25

    grid = _P0 // tp
    return pl.pallas_call(
        body,
        grid=(grid,),
        in_specs=[
            pl.BlockSpec((tp, 64), lambda i: (i, 0)),
            pl.BlockSpec((tp // 2, 128), lambda i: (i, 0)),
            pl.BlockSpec((tp // 2, 128), lambda i: (i, 0)),
            pl.BlockSpec((64, 128), lambda i: (0, 0)),
            pl.BlockSpec((128, 256), lambda i: (0, 0)),
            pl.BlockSpec((128, 256), lambda i: (0, 0)),
            pl.BlockSpec((256, 256), lambda i: (0, 0)),
            pl.BlockSpec((1, 256), lambda i: (0, 0)),
            pl.BlockSpec((1, 256), lambda i: (0, 0)),
            pl.BlockSpec((1, 256), lambda i: (0, 0)),
        ],
        out_specs=pl.BlockSpec((tp // 4, 128), lambda i: (i, 0)),
        out_shape=jax.ShapeDtypeStruct((_P1, 128), jnp.float32),
    )(x0, x1, x2, k0, k1, k2, s, b, g, be)


def _combine1_head(x0, x1, x2, k0, k1, k2, s, b, g, be, wh, bh, wo, bo):
    tp = 1024
    grid = _P1 // tp

    def body(x0r, x1r, x2r, k0r, k1r, k2r, sr, br, gr, ber,
             whr, bhr, wor, bor, outr, zsum):
        y = (jnp.dot(x0r[...], k0r[...], preferred_element_type=jnp.float32)
             + jnp.dot(x1r[...], k1r[...], preferred_element_type=jnp.float32)
             + jnp.dot(x2r[...], k2r[...], preferred_element_type=jnp.float32)
             + br[...])
        y = _ln_relu_lanes(y, sr[...], gr[...], ber[...])
        part = jnp.sum(y, axis=0, keepdims=True)

        @pl.when(pl.program_id(0) == 0)
        def _():
            zsum[...] = jnp.zeros_like(zsum)

        zsum[...] += part

        @pl.when(pl.program_id(0) == grid - 1)
        def _():
            z = zsum[...] * (1.0 / _P1)
            h = jnp.maximum(
                jnp.dot(z, whr[...], preferred_element_type=jnp.float32)
                + bhr[...], 0.0)
            outr[...] = (jnp.dot(h, wor[...],
                                 preferred_element_type=jnp.float32)
                         + bor[...])

    return pl.pallas_call(
        body,
        grid=(grid,),
        in_specs=[
            pl.BlockSpec((tp, 128), lambda i: (i, 0)),
            pl.BlockSpec((tp, 128), lambda i: (i, 0)),
            pl.BlockSpec((tp, 128), lambda i: (i, 0)),
            pl.BlockSpec((128, 128), lambda i: (0, 0)),
            pl.BlockSpec((128, 128), lambda i: (0, 0)),
            pl.BlockSpec((128, 128), lambda i: (0, 0)),
            pl.BlockSpec((128, 128), lambda i: (0, 0)),
            pl.BlockSpec((1, 128), lambda i: (0, 0)),
            pl.BlockSpec((1, 128), lambda i: (0, 0)),
            pl.BlockSpec((1, 128), lambda i: (0, 0)),
            pl.BlockSpec((128, _B * 256), lambda i: (0, 0)),
            pl.BlockSpec((1, _B * 256), lambda i: (0, 0)),
            pl.BlockSpec((_B * 256, _B * 128), lambda i: (0, 0)),
            pl.BlockSpec((1, _B * 128), lambda i: (0, 0)),
        ],
        out_specs=pl.BlockSpec((1, _B * 128), lambda i: (0, 0)),
        out_shape=jax.ShapeDtypeStruct((1, _B * 128), jnp.float32),
        scratch_shapes=[pltpu.VMEM((1, 128), jnp.float32)],
    )(x0, x1, x2, k0, k1, k2, s, b, g, be, wh, bh, wo, bo)


def kernel(x, ei0, w0_lap, ei1, w1_lap, cW0, cb0, g0, be0,
           cW1, cb1, g1, be1, Wh, bh, Wo, bo):
    lap0a = _make_sc_lap(_P0, _B * 16, emit_packed=True)
    lap0b = _make_sc_lap(_P0, _B * 16, table_packed=True)
    lap1a = _make_sc_lap(_P1, _B * 32, emit_packed=True)
    lap1b = _make_sc_lap(_P1, _B * 32, table_packed=True)

    xt = jnp.transpose(x, (1, 0, 2)).reshape(_P0, _B * 16)
    src0 = ei0[0].reshape(-1, 128)
    src1 = ei1[0].reshape(-1, 128)

    x1t, x1p = lap0a(xt, src0, w0_lap)
    y2t = lap0b(x1p.reshape(_P0, _B * 8), src0, w0_lap)

    eye = jnp.eye(_B, dtype=jnp.float32)
    eye8 = jnp.eye(2 * _B, dtype=jnp.float32)
    seg32 = jnp.kron(eye, jnp.full((32, 32), 1.0 / 32, jnp.float32))
    seg32x8 = jnp.kron(eye8, jnp.full((32, 32), 1.0 / 32, jnp.float32))

    h0t = _combine0(
        xt, x1t, y2t,
        jnp.kron(eye, cW0[0] - cW0[2]), jnp.kron(eye8, cW0[1]),
        jnp.kron(eye8, 2.0 * cW0[2]), seg32x8,
        jnp.tile(cb0, 2 * _B).reshape(1, -1),
        jnp.tile(g0, 2 * _B).reshape(1, -1),
        jnp.tile(be0, 2 * _B).reshape(1, -1))

    h1t, h1p = lap1a(h0t, src1, w1_lap)
    y2t1 = lap1b(h1p.reshape(_P1, _B * 16), src1, w1_lap)

    out = _combine1_head(
        h0t, h1t, y2t1,
        jnp.kron(eye, cW1[0] - cW1[2]), jnp.kron(eye, cW1[1]),
        jnp.kron(eye, 2.0 * cW1[2]), seg32,
        jnp.tile(cb1, _B).reshape(1, -1), jnp.tile(g1, _B).reshape(1, -1),
        jnp.tile(be1, _B).reshape(1, -1),
        jnp.kron(eye, Wh), jnp.tile(bh, _B).reshape(1, -1),
        jnp.kron(eye, Wo), jnp.tile(bo, _B).reshape(1, -1))
    return out.reshape(_B, 128)
